# baseline (speedup 1.0000x reference)
import jax
import jax.numpy as jnp
from jax import lax
from jax.experimental import pallas as pl
from jax.experimental.pallas import tpu as pltpu
from jax.experimental.pallas import tpu_sc as plsc

_BATCH = 16384
_NCOL = 4
_NWORKERS = 32
_FLAT = _BATCH * _NCOL
_PER_W = _FLAT // _NWORKERS
_VECS = _PER_W // 16


_GATHER_DNUMS = lax.GatherDimensionNumbers(
    offset_dims=(), collapsed_slice_dims=(0,), start_index_map=(0,))


def _take16(v, idx):
    return lax.gather(v, idx[:, None], _GATHER_DNUMS, (1,),
                      mode=lax.GatherScatterMode.PROMISE_IN_BOUNDS)


def _tec_body(x_hbm, w_hbm, b_hbm, out_hbm, x_v, o_v, w_v, b_v):
    wid = lax.axis_index("s") * 2 + lax.axis_index("c")
    base = wid * _PER_W
    pltpu.sync_copy(x_hbm.at[pl.ds(base, _PER_W)], x_v)
    pltpu.sync_copy(w_hbm, w_v)
    pltpu.sync_copy(b_hbm, b_v)

    scale = w_v[...] * jnp.float32(1.0 / 3.0)
    bias = b_v[...]
    lane = lax.iota(jnp.int32, 16)
    perm1 = lane ^ 1
    perm2 = lane ^ 2

    def body(k, carry):
        off = k * 16
        v = x_v[pl.ds(off, 16)]
        s1 = v + _take16(v, perm1)
        s = s1 + _take16(s1, perm2)
        o_v[pl.ds(off, 16)] = (s - v) * scale + bias
        return carry

    lax.fori_loop(0, _VECS, body, 0)
    pltpu.sync_copy(o_v, out_hbm.at[pl.ds(base, _PER_W)])


@jax.jit
def kernel(input, W, b):
    w16 = jnp.broadcast_to(jnp.reshape(W, ()), (16,)).astype(jnp.float32)
    b16 = jnp.broadcast_to(jnp.reshape(b, ()), (16,)).astype(jnp.float32)
    mesh = plsc.VectorSubcoreMesh(core_axis_name="c", subcore_axis_name="s")
    f = pl.kernel(
        _tec_body,
        mesh=mesh,
        out_type=jax.ShapeDtypeStruct((_FLAT,), jnp.float32),
        scratch_types=[
            pltpu.VMEM((_PER_W,), jnp.float32),
            pltpu.VMEM((_PER_W,), jnp.float32),
            pltpu.VMEM((16,), jnp.float32),
            pltpu.VMEM((16,), jnp.float32),
        ],
    )
    out = f(jnp.reshape(input, (_FLAT,)), w16, b16)
    return jnp.reshape(out, (_BATCH, _NCOL))


# speedup vs baseline: 1.3273x; 1.3273x over previous
import jax
import jax.numpy as jnp
from jax import lax
from jax.experimental import pallas as pl
from jax.experimental.pallas import tpu as pltpu
from jax.experimental.pallas import tpu_sc as plsc

_BATCH = 16384
_NCOL = 4
_NWORKERS = 32
_ROWS_PER_W = _BATCH // _NWORKERS
_GROUPS = _ROWS_PER_W // 16


def _tec_body(x_hbm, w_hbm, b_hbm, out_hbm, x_v, w_v, b_v):
    wid = lax.axis_index("s") * 2 + lax.axis_index("c")
    rbase = wid * _ROWS_PER_W
    pltpu.sync_copy(x_hbm.at[pl.ds(rbase, _ROWS_PER_W), :], x_v)
    pltpu.sync_copy(w_hbm, w_v)
    pltpu.sync_copy(b_hbm, b_v)

    scale = w_v[...] * jnp.float32(1.0 / 3.0)
    bias = b_v[...]
    lane = lax.iota(jnp.int32, 16)
    col_idx = [jnp.full((16,), c, jnp.int32) for c in range(_NCOL)]

    def group(g, carry):
        rows = g * 16 + lane
        cols = [plsc.load_gather(x_v, [rows, col_idx[c]]) for c in range(_NCOL)]
        s = (cols[0] + cols[1]) + (cols[2] + cols[3])
        for c in range(_NCOL):
            plsc.store_scatter(x_v, [rows, col_idx[c]],
                               (s - cols[c]) * scale + bias)
        return carry

    lax.fori_loop(0, _GROUPS, group, 0)
    pltpu.sync_copy(x_v, out_hbm.at[pl.ds(rbase, _ROWS_PER_W), :])


@jax.jit
def kernel(input, W, b):
    w16 = jnp.broadcast_to(jnp.reshape(W, ()), (16,)).astype(jnp.float32)
    b16 = jnp.broadcast_to(jnp.reshape(b, ()), (16,)).astype(jnp.float32)
    mesh = plsc.VectorSubcoreMesh(core_axis_name="c", subcore_axis_name="s")
    f = pl.kernel(
        _tec_body,
        mesh=mesh,
        out_type=jax.ShapeDtypeStruct((_BATCH, _NCOL), jnp.float32),
        compiler_params=pltpu.CompilerParams(needs_layout_passes=False),
        scratch_types=[
            pltpu.VMEM((_ROWS_PER_W, _NCOL), jnp.float32),
            pltpu.VMEM((16,), jnp.float32),
            pltpu.VMEM((16,), jnp.float32),
        ],
    )
    return f(input, w16, b16)


# speedup vs baseline: 2.4406x; 1.8388x over previous
import jax
import jax.numpy as jnp
from jax import lax
from jax.experimental import pallas as pl
from jax.experimental.pallas import tpu as pltpu
from jax.experimental.pallas import tpu_sc as plsc

_BATCH = 16384
_NNODE = 4
_LANES = 128
_TILES = _BATCH // _LANES
_NWORKERS = 32
_T_PER_W = _TILES // _NWORKERS


def _tec_body(x_hbm, w_hbm, b_hbm, out_hbm, x_v, w_v, b_v):
    wid = lax.axis_index("s") * 2 + lax.axis_index("c")
    t0 = wid * _T_PER_W
    pltpu.sync_copy(x_hbm.at[pl.ds(t0, _T_PER_W)], x_v)
    pltpu.sync_copy(w_hbm, w_v)
    pltpu.sync_copy(b_hbm, b_v)

    scale = w_v[...] * jnp.float32(1.0 / 3.0)
    bias = b_v[...]

    for t in range(_T_PER_W):
        for off in range(0, _LANES, 16):
            sl = pl.ds(off, 16)
            v = [x_v[t, r, sl] for r in range(_NNODE)]
            s = (v[0] + v[1]) + (v[2] + v[3])
            for r in range(_NNODE):
                x_v[t, r, sl] = (s - v[r]) * scale + bias

    pltpu.sync_copy(x_v, out_hbm.at[pl.ds(t0, _T_PER_W)])


@jax.jit
def kernel(input, W, b):
    xv = jnp.transpose(jnp.reshape(jnp.transpose(input), (_NNODE, _TILES, _LANES)),
                       (1, 0, 2))
    mesh = plsc.VectorSubcoreMesh(core_axis_name="c", subcore_axis_name="s")
    f = pl.kernel(
        _tec_body,
        mesh=mesh,
        out_type=jax.ShapeDtypeStruct((_TILES, _NNODE, _LANES), jnp.float32),
        compiler_params=pltpu.CompilerParams(needs_layout_passes=False),
        scratch_types=[
            pltpu.VMEM((_T_PER_W, _NNODE, _LANES), jnp.float32),
            pltpu.VMEM((16,), jnp.float32),
            pltpu.VMEM((16,), jnp.float32),
        ],
    )
    w16 = jnp.broadcast_to(jnp.reshape(W, ()), (16,)).astype(jnp.float32)
    b16 = jnp.broadcast_to(jnp.reshape(b, ()), (16,)).astype(jnp.float32)
    out = f(xv, w16, b16)
    return jnp.transpose(jnp.reshape(jnp.transpose(out, (1, 0, 2)),
                                     (_NNODE, _BATCH)))


# speedup vs baseline: 2.4562x; 1.0064x over previous
import jax
import jax.numpy as jnp
from jax import lax
from jax.experimental import pallas as pl
from jax.experimental.pallas import tpu as pltpu
from jax.experimental.pallas import tpu_sc as plsc

_BATCH = 16384
_NNODE = 4
_LANES = 128
_TILES = _BATCH // _LANES
_NWORKERS = 32
_T_PER_W = _TILES // _NWORKERS


def _tec_body(x_hbm, w_hbm, b_hbm, out_hbm, x_v, w_v, b_v):
    wid = lax.axis_index("s") * 2 + lax.axis_index("c")
    t0 = wid * _T_PER_W
    pltpu.sync_copy(x_hbm.at[pl.ds(t0, _T_PER_W)], x_v)
    pltpu.sync_copy(w_hbm, w_v)
    pltpu.sync_copy(b_hbm, b_v)

    scale = w_v[...] * jnp.float32(1.0 / 3.0)
    bias = b_v[...]

    def body(k, carry):
        t = k // 8
        sl = pl.ds((k % 8) * 16, 16)
        v = [x_v[t, r, sl] for r in range(_NNODE)]
        s = (v[0] + v[1]) + (v[2] + v[3])
        for r in range(_NNODE):
            x_v[t, r, sl] = (s - v[r]) * scale + bias
        return carry

    lax.fori_loop(0, _T_PER_W * 8, body, 0)

    pltpu.sync_copy(x_v, out_hbm.at[pl.ds(t0, _T_PER_W)])


@jax.jit
def kernel(input, W, b):
    xv = jnp.transpose(jnp.reshape(jnp.transpose(input), (_NNODE, _TILES, _LANES)),
                       (1, 0, 2))
    mesh = plsc.VectorSubcoreMesh(core_axis_name="c", subcore_axis_name="s")
    f = pl.kernel(
        _tec_body,
        mesh=mesh,
        out_type=jax.ShapeDtypeStruct((_TILES, _NNODE, _LANES), jnp.float32),
        compiler_params=pltpu.CompilerParams(needs_layout_passes=False),
        scratch_types=[
            pltpu.VMEM((_T_PER_W, _NNODE, _LANES), jnp.float32),
            pltpu.VMEM((16,), jnp.float32),
            pltpu.VMEM((16,), jnp.float32),
        ],
    )
    w16 = jnp.broadcast_to(jnp.reshape(W, ()), (16,)).astype(jnp.float32)
    b16 = jnp.broadcast_to(jnp.reshape(b, ()), (16,)).astype(jnp.float32)
    out = f(xv, w16, b16)
    return jnp.transpose(jnp.reshape(jnp.transpose(out, (1, 0, 2)),
                                     (_NNODE, _BATCH)))


# speedup vs baseline: 2.4699x; 1.0056x over previous
import jax
import jax.numpy as jnp
from jax import lax
from jax.experimental import pallas as pl
from jax.experimental.pallas import tpu as pltpu
from jax.experimental.pallas import tpu_sc as plsc

_BATCH = 16384
_NNODE = 4
_LANES = 128
_TILES = _BATCH // _LANES
_NWORKERS = 32
_T_PER_W = _TILES // _NWORKERS


def _tec_body(x_hbm, w_hbm, b_hbm, out_hbm, x_v, w_v, b_v):
    wid = lax.axis_index("s") * 2 + lax.axis_index("c")
    t0 = wid * _T_PER_W
    pltpu.sync_copy(x_hbm.at[pl.ds(t0, _T_PER_W)], x_v)
    pltpu.sync_copy(w_hbm, w_v)
    pltpu.sync_copy(b_hbm, b_v)

    scale = w_v[...] * jnp.float32(1.0 / 3.0)
    bias = b_v[...]

    def body(k, carry):
        t = k // 8
        sl = pl.ds((k % 8) * 16, 16)
        v = [x_v[t, r, sl] for r in range(_NNODE)]
        s = (v[0] + v[1]) + (v[2] + v[3])
        for r in range(_NNODE):
            x_v[t, r, sl] = (s - v[r]) * scale + bias
        return carry

    lax.fori_loop(0, _T_PER_W * 8, body, 0)

    pltpu.sync_copy(x_v, out_hbm.at[pl.ds(t0, _T_PER_W)])


@jax.jit
def kernel(input, W, b):
    xv = jnp.transpose(jnp.reshape(jnp.transpose(input), (_NNODE, _TILES, _LANES)),
                       (1, 0, 2))
    mesh = plsc.VectorSubcoreMesh(core_axis_name="c", subcore_axis_name="s")
    f = pl.kernel(
        _tec_body,
        mesh=mesh,
        out_type=jax.ShapeDtypeStruct((_TILES, _NNODE, _LANES), jnp.float32),
        compiler_params=pltpu.CompilerParams(needs_layout_passes=False, skip_device_barrier=True, disable_bounds_checks=True, disable_semaphore_checks=True),
        scratch_types=[
            pltpu.VMEM((_T_PER_W, _NNODE, _LANES), jnp.float32),
            pltpu.VMEM((16,), jnp.float32),
            pltpu.VMEM((16,), jnp.float32),
        ],
    )
    w16 = jnp.broadcast_to(jnp.reshape(W, ()), (16,)).astype(jnp.float32)
    b16 = jnp.broadcast_to(jnp.reshape(b, ()), (16,)).astype(jnp.float32)
    out = f(xv, w16, b16)
    return jnp.transpose(jnp.reshape(jnp.transpose(out, (1, 0, 2)),
                                     (_NNODE, _BATCH)))
